# baseline (speedup 1.0000x reference)
import functools

import jax
import jax.numpy as jnp
from jax import lax
from jax.experimental import pallas as pl
from jax.experimental.pallas import tpu as pltpu
from jax.experimental.pallas import tpu_sc as plsc

_BVS = 1000000
_BD = 64
_MD = 1024
_B, _S = 4, 2048
_N = _B * _S
_NC, _NS, _L = 2, 16, 16
_NW = _NC * _NS
_CHUNK = _N // _NW
_PAD = 8


def _sc_hash_gather(ids_pad, table):
    mesh = plsc.VectorSubcoreMesh(core_axis_name="c", subcore_axis_name="s")

    @functools.partial(
        pl.kernel,
        mesh=mesh,
        compiler_params=pltpu.CompilerParams(use_tc_tiling_on_sc=False),
        out_type=jax.ShapeDtypeStruct((_N, _BD), jnp.float32),
        scratch_types=[
            pltpu.VMEM((_CHUNK + _PAD,), jnp.int32),
            pltpu.VMEM((2, 128), jnp.int32),
            pltpu.VMEM((_CHUNK, _BD), jnp.float32),
            pltpu.SemaphoreType.DMA,
        ],
    )
    def run(ids_hbm, table_hbm, out_hbm, buf_v, idx_v, rows_v, sem):
        wid = lax.axis_index("s") * _NC + lax.axis_index("c")
        base = wid * _CHUNK
        pltpu.sync_copy(ids_hbm.at[pl.ds(base, _CHUNK + _PAD)], buf_v)

        lanes = lax.iota(jnp.int32, _L)
        for i in range(_CHUNK // _L):
            off = _PAD + i * _L
            cur = buf_v[pl.ds(off, _L)]
            prev = buf_v[pl.ds(off - 1, _L)]
            h = lax.rem(jnp.bitwise_xor(cur * 36313, prev * 27191),
                        jnp.int32(_BVS - 1))
            pos = base + i * _L + lanes
            h = jnp.where((pos & (_S - 1)) == 0, jnp.int32(_BVS - 1), h)
            idx_v[i // 8, pl.ds((i % 8) * _L, _L)] = h

        cps = [
            pltpu.async_copy(table_hbm.at[idx_v.at[r]],
                             rows_v.at[pl.ds(r * 128, 128)], sem)
            for r in range(2)
        ]
        for cp in cps:
            cp.wait()
        pltpu.sync_copy(rows_v, out_hbm.at[pl.ds(base, _CHUNK)])

    return run(ids_pad, table)


def _tc_project(x, w, scale):
    blk = 1024

    def body(s_ref, x_ref, w_ref, o_ref):
        acc = lax.dot_general(x_ref[...], w_ref[...],
                              (((1,), (1,)), ((), ())),
                              preferred_element_type=jnp.float32)
        o_ref[...] = acc * s_ref[0, 0]

    return pl.pallas_call(
        body,
        grid=(_N // blk,),
        in_specs=[
            pl.BlockSpec(memory_space=pltpu.SMEM),
            pl.BlockSpec((blk, _BD), lambda i: (i, 0)),
            pl.BlockSpec((_MD, _BD), lambda i: (0, 0)),
        ],
        out_specs=pl.BlockSpec((blk, _MD), lambda i: (i, 0)),
        out_shape=jax.ShapeDtypeStruct((_N, _MD), jnp.float32),
    )(scale, x, w)


def kernel(ids, embed_weight, proj_weight, scale):
    ids_flat = ids.astype(jnp.int32).reshape(_N)
    ids_pad = jnp.concatenate([jnp.zeros((_PAD,), jnp.int32), ids_flat])
    rows = _sc_hash_gather(ids_pad, embed_weight)
    out = _tc_project(rows, proj_weight,
                      scale.astype(jnp.float32).reshape(1, 1))
    return out.reshape(_B, _S, _MD)
